# baseline (speedup 1.0000x reference)
import functools

import jax
import jax.numpy as jnp
from jax import lax
from jax.experimental import pallas as pl
from jax.experimental.pallas import tpu as pltpu
from jax.experimental.pallas import tpu_sc as plsc

_info = plsc.get_sparse_core_info()
_NC = _info.num_cores
_NS = _info.num_subcores
_NW = _NC * _NS

_NBUF = 4
_N_CHUNKS = 16


@functools.lru_cache(maxsize=None)
def _make_gather(V, D, B):
    assert B % _NW == 0
    b_per_w = B // _NW
    assert b_per_w % _N_CHUNKS == 0
    chunk = b_per_w // _N_CHUNKS
    assert chunk % 8 == 0
    mesh = plsc.VectorSubcoreMesh(core_axis_name="c", subcore_axis_name="s")

    @functools.partial(
        pl.kernel,
        mesh=mesh,
        out_type=jax.ShapeDtypeStruct((B, D), jnp.float32),
        scratch_types=[
            pltpu.VMEM((b_per_w,), jnp.int32),
            [pltpu.VMEM((chunk, D), jnp.float32) for _ in range(_NBUF)],
            [pltpu.SemaphoreType.DMA for _ in range(_NBUF)],
            [pltpu.SemaphoreType.DMA for _ in range(_NBUF)],
        ],
        compiler_params=pltpu.CompilerParams(use_tc_tiling_on_sc=False),
    )
    def gather_kernel(table_hbm, idx_hbm, out_hbm, idx_v, bufs, sgs, sws):
        wid = lax.axis_index("s") * _NC + lax.axis_index("c")
        base = wid * b_per_w

        pltpu.sync_copy(idx_hbm.at[pl.ds(base, b_per_w)], idx_v)

        gathers = {}
        writes = {}

        def start_gather(c):
            b = c % _NBUF
            gathers[c] = pltpu.async_copy(
                table_hbm.at[idx_v.at[pl.ds(c * chunk, chunk)]], bufs[b],
                sgs[b])

        def start_write(c):
            b = c % _NBUF
            writes[c] = pltpu.async_copy(
                bufs[b], out_hbm.at[pl.ds(base + c * chunk, chunk)], sws[b])

        for c in range(_NBUF):
            start_gather(c)
        for c in range(_N_CHUNKS):
            gathers[c].wait()
            start_write(c)
            if c + _NBUF < _N_CHUNKS:
                writes[c].wait()
                start_gather(c + _NBUF)
        for c in range(_N_CHUNKS - _NBUF, _N_CHUNKS):
            writes[c].wait()

    return gather_kernel


def kernel(cat_idx, table):
    batch, fields = cat_idx.shape
    V, D = table.shape
    B = batch * fields
    idx_flat = cat_idx.reshape(B).astype(jnp.int32)
    out = _make_gather(V, D, B)(table, idx_flat)
    return out.reshape(batch, fields, D)


# speedup vs baseline: 1.0058x; 1.0058x over previous
import functools

import jax
import jax.numpy as jnp
from jax import lax
from jax.experimental import pallas as pl
from jax.experimental.pallas import tpu as pltpu
from jax.experimental.pallas import tpu_sc as plsc

_info = plsc.get_sparse_core_info()
_NC = _info.num_cores
_NS = _info.num_subcores
_NW = _NC * _NS

_NBUF = 4
_N_CHUNKS = 16

_VB = 2048


def _repack_body(t32_ref, rep_ref):
    x = t32_ref[...]
    xT = jnp.transpose(x, (1, 0))
    x3 = xT.reshape(_VB // 4, 4, 32)
    rep_ref[...] = jnp.concatenate(
        [x3[:, a, :] for a in range(4)], axis=1)


@functools.lru_cache(maxsize=None)
def _make_repack(V):
    n_blocks = (V + _VB - 1) // _VB
    rep_rows = V * 32 // 128
    return pl.pallas_call(
        _repack_body,
        grid=(n_blocks,),
        in_specs=[pl.BlockSpec((32, _VB), lambda j: (0, j))],
        out_specs=pl.BlockSpec((_VB // 4, 128), lambda j: (j, 0)),
        out_shape=jax.ShapeDtypeStruct((rep_rows, 128), jnp.float32),
    )


@functools.lru_cache(maxsize=None)
def _make_gather(V, D, B):
    assert B % _NW == 0
    b_per_w = B // _NW
    assert b_per_w % _N_CHUNKS == 0
    chunk = b_per_w // _N_CHUNKS
    assert chunk % 8 == 0
    mesh = plsc.VectorSubcoreMesh(core_axis_name="c", subcore_axis_name="s")

    @functools.partial(
        pl.kernel,
        mesh=mesh,
        out_type=jax.ShapeDtypeStruct((B, D), jnp.float32),
        scratch_types=[
            pltpu.VMEM((b_per_w,), jnp.int32),
            [pltpu.VMEM((chunk, D), jnp.float32) for _ in range(_NBUF)],
            [pltpu.SemaphoreType.DMA for _ in range(_NBUF)],
            [pltpu.SemaphoreType.DMA for _ in range(_NBUF)],
        ],
        compiler_params=pltpu.CompilerParams(use_tc_tiling_on_sc=False),
    )
    def gather_kernel(table_hbm, idx_hbm, out_hbm, idx_v, bufs, sgs, sws):
        wid = lax.axis_index("s") * _NC + lax.axis_index("c")
        base = wid * b_per_w

        pltpu.sync_copy(idx_hbm.at[pl.ds(base, b_per_w)], idx_v)

        gathers = {}
        writes = {}

        def start_gather(c):
            b = c % _NBUF
            gathers[c] = pltpu.async_copy(
                table_hbm.at[idx_v.at[pl.ds(c * chunk, chunk)]], bufs[b],
                sgs[b])

        def start_write(c):
            b = c % _NBUF
            writes[c] = pltpu.async_copy(
                bufs[b], out_hbm.at[pl.ds(base + c * chunk, chunk)], sws[b])

        for c in range(_NBUF):
            start_gather(c)
        for c in range(_N_CHUNKS):
            gathers[c].wait()
            start_write(c)
            if c + _NBUF < _N_CHUNKS:
                writes[c].wait()
                start_gather(c + _NBUF)
        for c in range(_N_CHUNKS - _NBUF, _N_CHUNKS):
            writes[c].wait()

    return gather_kernel


def kernel(cat_idx, table):
    batch, fields = cat_idx.shape
    V, D = table.shape
    B = batch * fields
    idx_flat = cat_idx.reshape(B).astype(jnp.int32)
    rep = _make_repack(V)(table.T)
    table_lin = rep.reshape(V, D)
    out = _make_gather(V, D, B)(table_lin, idx_flat)
    return out.reshape(batch, fields, D)


# speedup vs baseline: 1.4130x; 1.4049x over previous
import functools

import jax
import jax.numpy as jnp
from jax import lax
from jax.experimental import pallas as pl
from jax.experimental.pallas import tpu as pltpu
from jax.experimental.pallas import tpu_sc as plsc

_info = plsc.get_sparse_core_info()
_NC = _info.num_cores
_NS = _info.num_subcores
_NW = _NC * _NS

_NBUF = 4
_N_CHUNKS = 16

_VBO = 2048


def _repack_body(x_ref, rep_ref):
    x = x_ref[...]
    rep_ref[...] = jnp.concatenate(
        [jnp.transpose(x[:, a * _VBO:(a + 1) * _VBO], (1, 0))
         for a in range(4)], axis=1)


@functools.lru_cache(maxsize=None)
def _make_repack(V):
    n_blocks = -(-V // (4 * _VBO))
    rep_rows = n_blocks * _VBO

    return pl.pallas_call(
        _repack_body,
        grid=(n_blocks,),
        in_specs=[pl.BlockSpec((32, 4 * _VBO), lambda j: (0, j))],
        out_specs=pl.BlockSpec((_VBO, 128), lambda j: (j, 0)),
        out_shape=jax.ShapeDtypeStruct((rep_rows, 128), jnp.float32),
    )


@functools.lru_cache(maxsize=None)
def _make_gather(V, D, B):
    assert B % _NW == 0
    b_per_w = B // _NW
    assert b_per_w % _N_CHUNKS == 0
    chunk = b_per_w // _N_CHUNKS
    assert chunk % 8 == 0 and b_per_w % 16 == 0
    mesh = plsc.VectorSubcoreMesh(core_axis_name="c", subcore_axis_name="s")

    @functools.partial(
        pl.kernel,
        mesh=mesh,
        out_type=jax.ShapeDtypeStruct((B, D), jnp.float32),
        scratch_types=[
            pltpu.VMEM((b_per_w,), jnp.int32),
            [pltpu.VMEM((chunk, D), jnp.float32) for _ in range(_NBUF)],
            [pltpu.SemaphoreType.DMA for _ in range(_NBUF)],
            [pltpu.SemaphoreType.DMA for _ in range(_NBUF)],
        ],
        compiler_params=pltpu.CompilerParams(use_tc_tiling_on_sc=False),
    )
    def gather_kernel(table_hbm, idx_hbm, out_hbm, idx_v, bufs, sgs, sws):
        wid = lax.axis_index("s") * _NC + lax.axis_index("c")
        base = wid * b_per_w

        pltpu.sync_copy(idx_hbm.at[pl.ds(base, b_per_w)], idx_v)

        gathers = {}
        writes = {}

        def start_gather(c):
            b = c % _NBUF
            gathers[c] = pltpu.async_copy(
                table_hbm.at[idx_v.at[pl.ds(c * chunk, chunk)]], bufs[b],
                sgs[b])

        def start_write(c):
            b = c % _NBUF
            writes[c] = pltpu.async_copy(
                bufs[b], out_hbm.at[pl.ds(base + c * chunk, chunk)], sws[b])

        for c in range(_NBUF):
            start_gather(c)
        for c in range(_N_CHUNKS):
            gathers[c].wait()
            start_write(c)
            if c + _NBUF < _N_CHUNKS:
                writes[c].wait()
                start_gather(c + _NBUF)
        for c in range(_N_CHUNKS - _NBUF, _N_CHUNKS):
            writes[c].wait()

    return gather_kernel


def kernel(cat_idx, table):
    batch, fields = cat_idx.shape
    V, D = table.shape
    B = batch * fields
    idx_flat = cat_idx.reshape(B).astype(jnp.int32)
    idx_perm = ((((idx_flat >> 13) << 11) | (idx_flat & 2047)) << 2) | (
        (idx_flat >> 11) & 3)
    t32 = table.T
    rep = _make_repack(V)(t32)
    Vp = rep.shape[0] * 128 // D
    table_lin = rep.reshape(Vp, D)
    out = _make_gather(Vp, D, B)(table_lin, idx_perm)
    return out.reshape(batch, fields, D)


# speedup vs baseline: 1.5644x; 1.1072x over previous
import functools

import jax
import jax.numpy as jnp
from jax import lax
from jax.experimental import pallas as pl
from jax.experimental.pallas import tpu as pltpu
from jax.experimental.pallas import tpu_sc as plsc

_info = plsc.get_sparse_core_info()
_NC = _info.num_cores
_NS = _info.num_subcores
_NW = _NC * _NS

_NBUF = 4
_N_CHUNKS = 16

_VBO = 2048


def _repack_body(x_ref, rep_ref):
    x = x_ref[...]
    rep_ref[...] = jnp.concatenate(
        [jnp.transpose(x[:, a * _VBO:(a + 1) * _VBO], (1, 0))
         for a in range(4)], axis=1)


@functools.lru_cache(maxsize=None)
def _make_repack(V):
    n_blocks = -(-V // (4 * _VBO))
    rep_rows = n_blocks * _VBO

    return pl.pallas_call(
        _repack_body,
        grid=(n_blocks,),
        in_specs=[pl.BlockSpec((32, 4 * _VBO), lambda j: (0, j))],
        out_specs=pl.BlockSpec((_VBO, 128), lambda j: (j, 0)),
        out_shape=jax.ShapeDtypeStruct((rep_rows, 128), jnp.float32),
    )


@functools.lru_cache(maxsize=None)
def _make_gather(V, D, B):
    assert B % _NW == 0
    b_per_w = B // _NW
    assert b_per_w % _N_CHUNKS == 0
    chunk = b_per_w // _N_CHUNKS
    assert chunk % 8 == 0 and b_per_w % 16 == 0
    mesh = plsc.VectorSubcoreMesh(core_axis_name="c", subcore_axis_name="s")

    @functools.partial(
        pl.kernel,
        mesh=mesh,
        out_type=jax.ShapeDtypeStruct((B, D), jnp.float32),
        scratch_types=[
            pltpu.VMEM((b_per_w,), jnp.int32),
            [pltpu.VMEM((chunk, D), jnp.float32) for _ in range(_NBUF)],
            [pltpu.SemaphoreType.DMA for _ in range(_NBUF)],
            [pltpu.SemaphoreType.DMA for _ in range(_NBUF)],
        ],
        compiler_params=pltpu.CompilerParams(use_tc_tiling_on_sc=False),
    )
    def gather_kernel(table_hbm, idx_hbm, out_hbm, idx_v, bufs, sgs, sws):
        wid = lax.axis_index("s") * _NC + lax.axis_index("c")
        base = wid * b_per_w

        pltpu.sync_copy(idx_hbm.at[pl.ds(base, b_per_w)], idx_v)

        gathers = {}
        writes = {}

        def start_gather(c):
            b = c % _NBUF
            gathers[c] = pltpu.async_copy(
                table_hbm.at[idx_v.at[pl.ds(c * chunk, chunk)]], bufs[b],
                sgs[b])

        def start_write(c):
            b = c % _NBUF
            writes[c] = pltpu.async_copy(
                bufs[b], out_hbm.at[pl.ds(base + c * chunk, chunk)], sws[b])

        for c in range(_NBUF):
            start_gather(c)
        for c in range(_N_CHUNKS):
            gathers[c].wait()
            start_write(c)
            if c + _NBUF < _N_CHUNKS:
                writes[c].wait()
                start_gather(c + _NBUF)
        for c in range(_N_CHUNKS - _NBUF, _N_CHUNKS):
            writes[c].wait()

    return gather_kernel


def kernel(cat_idx, table):
    batch, fields = cat_idx.shape
    V, D = table.shape
    B = batch * fields
    idx_flat = cat_idx.T.reshape(B).astype(jnp.int32)
    idx_perm = ((((idx_flat >> 13) << 11) | (idx_flat & 2047)) << 2) | (
        (idx_flat >> 11) & 3)
    t32 = table.T
    rep = _make_repack(V)(t32)
    Vp = rep.shape[0] * 128 // D
    table_lin = rep.reshape(Vp, D)
    out = _make_gather(Vp, D, B)(table_lin, idx_perm)
    return jnp.transpose(out.reshape(fields, batch, D), (1, 0, 2))


# speedup vs baseline: 2.1617x; 1.3818x over previous
import functools

import jax
import jax.numpy as jnp
from jax import lax
from jax.experimental import pallas as pl
from jax.experimental.pallas import tpu as pltpu
from jax.experimental.pallas import tpu_sc as plsc

_info = plsc.get_sparse_core_info()
_NC = _info.num_cores
_NS = _info.num_subcores
_NW = _NC * _NS

_NBUF = 4
_N_CHUNKS = 16

_VBO = 2048


def _repack_body(x_ref, rep_ref):
    x = x_ref[...]
    rep_ref[...] = jnp.concatenate(
        [jnp.transpose(x[:, a * _VBO:(a + 1) * _VBO], (1, 0))
         for a in range(4)], axis=1)


@functools.lru_cache(maxsize=None)
def _make_repack(V):
    n_blocks = -(-V // (4 * _VBO))
    rep_rows = n_blocks * _VBO

    return pl.pallas_call(
        _repack_body,
        grid=(n_blocks,),
        in_specs=[pl.BlockSpec((32, 4 * _VBO), lambda j: (0, j))],
        out_specs=pl.BlockSpec((_VBO, 128), lambda j: (j, 0)),
        out_shape=jax.ShapeDtypeStruct((rep_rows, 128), jnp.float32),
    )


@functools.lru_cache(maxsize=None)
def _make_gather(V, D, B):
    assert B % _NW == 0
    b_per_w = B // _NW
    assert b_per_w % _N_CHUNKS == 0
    chunk = b_per_w // _N_CHUNKS
    assert chunk % 8 == 0 and b_per_w % 16 == 0
    mesh = plsc.VectorSubcoreMesh(core_axis_name="c", subcore_axis_name="s")

    @functools.partial(
        pl.kernel,
        mesh=mesh,
        out_type=jax.ShapeDtypeStruct((B, 128), jnp.float32),
        scratch_types=[
            pltpu.VMEM((b_per_w,), jnp.int32),
            [pltpu.VMEM((chunk, D), jnp.float32) for _ in range(_NBUF)],
            [pltpu.SemaphoreType.DMA for _ in range(_NBUF)],
            [pltpu.SemaphoreType.DMA for _ in range(_NBUF)],
        ],
        compiler_params=pltpu.CompilerParams(use_tc_tiling_on_sc=False),
    )
    def gather_kernel(table_hbm, idx_hbm, out_hbm, idx_v, bufs, sgs, sws):
        wid = lax.axis_index("s") * _NC + lax.axis_index("c")
        base = wid * b_per_w

        pltpu.sync_copy(idx_hbm.at[pl.ds(base, b_per_w)], idx_v)

        gathers = {}
        writes = {}

        def start_gather(c):
            b = c % _NBUF
            gathers[c] = pltpu.async_copy(
                table_hbm.at[idx_v.at[pl.ds(c * chunk, chunk)]], bufs[b],
                sgs[b])

        def start_write(c):
            b = c % _NBUF
            writes[c] = pltpu.async_copy(
                bufs[b],
                out_hbm.at[pl.ds(base + c * chunk, chunk), pl.ds(0, D)],
                sws[b])

        for c in range(_NBUF):
            start_gather(c)
        for c in range(_N_CHUNKS):
            gathers[c].wait()
            start_write(c)
            if c + _NBUF < _N_CHUNKS:
                writes[c].wait()
                start_gather(c + _NBUF)
        for c in range(_N_CHUNKS - _NBUF, _N_CHUNKS):
            writes[c].wait()

    return gather_kernel


def kernel(cat_idx, table):
    batch, fields = cat_idx.shape
    V, D = table.shape
    B = batch * fields
    idx_flat = cat_idx.T.reshape(B).astype(jnp.int32)
    idx_perm = ((((idx_flat >> 13) << 11) | (idx_flat & 2047)) << 2) | (
        (idx_flat >> 11) & 3)
    t32 = table.T
    rep = _make_repack(V)(t32)
    Vp = rep.shape[0] * 128 // D
    table_lin = rep.reshape(Vp, D)
    out_pad = _make_gather(Vp, D, B)(table_lin, idx_perm)
    out3 = out_pad.reshape(fields, batch, 128)[:, :, :D]
    return jnp.transpose(out3, (1, 0, 2))
